# baseline (speedup 1.0000x reference)
import functools

import jax
import jax.numpy as jnp
from jax import lax
from jax.experimental import pallas as pl
from jax.experimental.pallas import tpu as pltpu
from jax.experimental.pallas import tpu_sc as plsc

_NC = 2
_NS = 16
_NW = _NC * _NS
_L = 16

_B, _H, _W = 32, 512, 512
_R = 14
_BT = _B - _R
_ROWS = 32
_CPR = _H // _ROWS
_NCHUNKS = _R * _CPR
_CPW = _NCHUNKS // _NW
_U = 4
_VPC = _ROWS * _W // _L

assert _NCHUNKS % _NW == 0


def _sc_body(sal_hbm, fix_hbm, out_hbm, sbuf, fbuf, part,
             sem_s0, sem_s1, sem_f0, sem_f1):
    wid = lax.axis_index("s") * _NC + lax.axis_index("c")
    sems = (sem_s0, sem_s1)
    semf = (sem_f0, sem_f1)

    def copies(i, b):
        g = wid + i * _NW
        row = _BT + g // _CPR
        rows = pl.ds((g % _CPR) * _ROWS, _ROWS)
        return (
            pltpu.make_async_copy(sal_hbm.at[row, rows], sbuf.at[b], sems[b]),
            pltpu.make_async_copy(fix_hbm.at[row, rows], fbuf.at[b], semf[b]),
        )

    for b in range(2):
        for cp in copies(b, b):
            cp.start()

    zero = jnp.zeros((_L,), jnp.float32)
    accs0 = tuple((zero, zero, zero, zero) for _ in range(_U))

    def chunk_compute(b, accs):
        def step(j, accs):
            out = []
            for u in range(_U):
                a_sum, a_sq, a_m, a_c = accs[u]
                k = j * _U + u
                r = k >> 5
                col = (k & 31) * _L
                v = sbuf[b, r, pl.ds(col, _L)]
                f = fbuf[b, r, pl.ds(col, _L)]
                m = f > 0.1
                a_sum = a_sum + v
                a_sq = a_sq + v * v
                a_m = a_m + jnp.where(m, v, 0.0)
                a_c = a_c + jnp.where(m, 1.0, 0.0)
                out.append((a_sum, a_sq, a_m, a_c))
            return tuple(out)

        return plsc.parallel_loop(0, _VPC // _U, unroll=2, carry=accs)(step)

    accs = accs0
    for i in range(_CPW):
        b = i % 2
        for cp in copies(i, b):
            cp.wait()
        accs = chunk_compute(b, accs)
        if i + 2 < _CPW:
            for cp in copies(i + 2, b):
                cp.start()
    a_sum, a_sq, a_m, a_c = accs[0]
    for u in range(1, _U):
        a_sum = a_sum + accs[u][0]
        a_sq = a_sq + accs[u][1]
        a_m = a_m + accs[u][2]
        a_c = a_c + accs[u][3]

    part[0, :] = a_sum
    part[1, :] = a_sq
    part[2, :] = a_m
    part[3, :] = a_c
    pltpu.sync_copy(part, out_hbm.at[wid])


_sc_reduce = functools.partial(
    pl.kernel,
    mesh=plsc.VectorSubcoreMesh(core_axis_name="c", subcore_axis_name="s"),
    out_type=jax.ShapeDtypeStruct((_NW, 4, _L), jnp.float32),
    scratch_types=[
        pltpu.VMEM((2, _ROWS, _W), jnp.float32),
        pltpu.VMEM((2, _ROWS, _W), jnp.float32),
        pltpu.VMEM((4, _L), jnp.float32),
        pltpu.SemaphoreType.DMA,
        pltpu.SemaphoreType.DMA,
        pltpu.SemaphoreType.DMA,
        pltpu.SemaphoreType.DMA,
    ],
)(_sc_body)


def _tc_body(sal_ref, fix_ref, out_ref):
    i = pl.program_id(0) + pl.program_id(1)
    s = sal_ref[...]
    f = fix_ref[...]
    m = f > 0.1
    ssum = jnp.sum(s)
    ssq = jnp.sum(s * s)
    msum = jnp.sum(jnp.where(m, s, 0.0))
    cnt = jnp.sum(jnp.where(m, 1.0, 0.0))

    @pl.when(i == 0)
    def _init():
        out_ref[0] = 0.0
        out_ref[1] = 0.0
        out_ref[2] = 0.0
        out_ref[3] = 0.0

    out_ref[0] += ssum
    out_ref[1] += ssq
    out_ref[2] += msum
    out_ref[3] += cnt


def kernel(sal_map, fix):
    n = sal_map.size
    tc_partials = pl.pallas_call(
        _tc_body,
        grid=(_B, 4),
        in_specs=[
            pl.BlockSpec((1, _H // 4, _W), lambda i, j: (i, j, 0)),
            pl.BlockSpec((1, _H // 4, _W), lambda i, j: (i, j, 0)),
        ],
        out_specs=pl.BlockSpec(memory_space=pltpu.SMEM),
        out_shape=jax.ShapeDtypeStruct((4,), jnp.float32),
    )(sal_map, fix)
    sums = tc_partials
    ssum, ssq, msum, cnt = sums[0], sums[1], sums[2], sums[3]
    nf = jnp.float32(n)
    mean = ssum / nf
    var = (ssq - nf * mean * mean) / (nf - 1.0)
    std = jnp.sqrt(var)
    return (msum - cnt * mean) / (std * cnt)


# speedup vs baseline: 2.8280x; 2.8280x over previous
import functools

import jax
import jax.numpy as jnp
from jax import lax
from jax.experimental import pallas as pl
from jax.experimental.pallas import tpu as pltpu
from jax.experimental.pallas import tpu_sc as plsc

_NC = 2
_NS = 16
_NW = _NC * _NS
_L = 16

_B, _H, _W = 32, 512, 512
_R = 14
_BT = _B - _R
_ROWS = 32
_CPR = _H // _ROWS
_NCHUNKS = _R * _CPR
_CPW = _NCHUNKS // _NW
_U = 4
_VPC = _ROWS * _W // _L

assert _NCHUNKS % _NW == 0


def _sc_body(sal_hbm, fix_hbm, out_hbm, sbuf, fbuf, part,
             sem_s0, sem_s1, sem_f0, sem_f1):
    wid = lax.axis_index("s") * _NC + lax.axis_index("c")
    sems = (sem_s0, sem_s1)
    semf = (sem_f0, sem_f1)

    def copies(i, b):
        g = wid + i * _NW
        row = _BT + g // _CPR
        rows = pl.ds((g % _CPR) * _ROWS, _ROWS)
        return (
            pltpu.make_async_copy(sal_hbm.at[row, rows], sbuf.at[b], sems[b]),
            pltpu.make_async_copy(fix_hbm.at[row, rows], fbuf.at[b], semf[b]),
        )

    for b in range(2):
        for cp in copies(b, b):
            cp.start()

    zero = jnp.zeros((_L,), jnp.float32)
    accs0 = tuple((zero, zero, zero, zero) for _ in range(_U))

    def chunk_compute(b, accs):
        def step(j, accs):
            out = []
            for u in range(_U):
                a_sum, a_sq, a_m, a_c = accs[u]
                k = j * _U + u
                r = k >> 5
                col = (k & 31) * _L
                v = sbuf[b, r, pl.ds(col, _L)]
                f = fbuf[b, r, pl.ds(col, _L)]
                m = f > 0.1
                a_sum = a_sum + v
                a_sq = a_sq + v * v
                a_m = a_m + jnp.where(m, v, 0.0)
                a_c = a_c + jnp.where(m, 1.0, 0.0)
                out.append((a_sum, a_sq, a_m, a_c))
            return tuple(out)

        return plsc.parallel_loop(0, _VPC // _U, unroll=2, carry=accs)(step)

    accs = accs0
    for i in range(_CPW):
        b = i % 2
        for cp in copies(i, b):
            cp.wait()
        accs = chunk_compute(b, accs)
        if i + 2 < _CPW:
            for cp in copies(i + 2, b):
                cp.start()
    a_sum, a_sq, a_m, a_c = accs[0]
    for u in range(1, _U):
        a_sum = a_sum + accs[u][0]
        a_sq = a_sq + accs[u][1]
        a_m = a_m + accs[u][2]
        a_c = a_c + accs[u][3]

    part[0, :] = a_sum
    part[1, :] = a_sq
    part[2, :] = a_m
    part[3, :] = a_c
    pltpu.sync_copy(part, out_hbm.at[wid])


_sc_reduce = functools.partial(
    pl.kernel,
    mesh=plsc.VectorSubcoreMesh(core_axis_name="c", subcore_axis_name="s"),
    out_type=jax.ShapeDtypeStruct((_NW, 4, _L), jnp.float32),
    scratch_types=[
        pltpu.VMEM((2, _ROWS, _W), jnp.float32),
        pltpu.VMEM((2, _ROWS, _W), jnp.float32),
        pltpu.VMEM((4, _L), jnp.float32),
        pltpu.SemaphoreType.DMA,
        pltpu.SemaphoreType.DMA,
        pltpu.SemaphoreType.DMA,
        pltpu.SemaphoreType.DMA,
    ],
)(_sc_body)


def _tc_body(sal_ref, fix_ref, out_ref):
    i = pl.program_id(0) + pl.program_id(1)
    s = sal_ref[...]
    f = fix_ref[...]
    m = f > 0.1
    ssum = jnp.sum(s)
    ssq = jnp.sum(s * s)
    msum = jnp.sum(jnp.where(m, s, 0.0))
    cnt = jnp.sum(jnp.where(m, 1.0, 0.0))

    @pl.when(i == 0)
    def _init():
        out_ref[0] = 0.0
        out_ref[1] = 0.0
        out_ref[2] = 0.0
        out_ref[3] = 0.0

    out_ref[0] += ssum
    out_ref[1] += ssq
    out_ref[2] += msum
    out_ref[3] += cnt


def kernel(sal_map, fix):
    n = sal_map.size
    tc_partials = pl.pallas_call(
        _tc_body,
        grid=(_B // 2, 1),
        in_specs=[
            pl.BlockSpec((2, _H, _W), lambda i, j: (i, 0, 0)),
            pl.BlockSpec((2, _H, _W), lambda i, j: (i, 0, 0)),
        ],
        out_specs=pl.BlockSpec(memory_space=pltpu.SMEM),
        out_shape=jax.ShapeDtypeStruct((4,), jnp.float32),
    )(sal_map, fix)
    sums = tc_partials
    ssum, ssq, msum, cnt = sums[0], sums[1], sums[2], sums[3]
    nf = jnp.float32(n)
    mean = ssum / nf
    var = (ssq - nf * mean * mean) / (nf - 1.0)
    std = jnp.sqrt(var)
    return (msum - cnt * mean) / (std * cnt)


# speedup vs baseline: 3.2134x; 1.1362x over previous
import functools

import jax
import jax.numpy as jnp
from jax import lax
from jax.experimental import pallas as pl
from jax.experimental.pallas import tpu as pltpu
from jax.experimental.pallas import tpu_sc as plsc

_NC = 2
_NS = 16
_NW = _NC * _NS
_L = 16

_B, _H, _W = 32, 512, 512
_R = 14
_BT = _B - _R
_ROWS = 32
_CPR = _H // _ROWS
_NCHUNKS = _R * _CPR
_CPW = _NCHUNKS // _NW
_U = 4
_VPC = _ROWS * _W // _L

assert _NCHUNKS % _NW == 0


def _sc_body(sal_hbm, fix_hbm, out_hbm, sbuf, fbuf, part,
             sem_s0, sem_s1, sem_f0, sem_f1):
    wid = lax.axis_index("s") * _NC + lax.axis_index("c")
    sems = (sem_s0, sem_s1)
    semf = (sem_f0, sem_f1)

    def copies(i, b):
        g = wid + i * _NW
        row = _BT + g // _CPR
        rows = pl.ds((g % _CPR) * _ROWS, _ROWS)
        return (
            pltpu.make_async_copy(sal_hbm.at[row, rows], sbuf.at[b], sems[b]),
            pltpu.make_async_copy(fix_hbm.at[row, rows], fbuf.at[b], semf[b]),
        )

    for b in range(2):
        for cp in copies(b, b):
            cp.start()

    zero = jnp.zeros((_L,), jnp.float32)
    accs0 = tuple((zero, zero, zero, zero) for _ in range(_U))

    def chunk_compute(b, accs):
        def step(j, accs):
            out = []
            for u in range(_U):
                a_sum, a_sq, a_m, a_c = accs[u]
                k = j * _U + u
                r = k >> 5
                col = (k & 31) * _L
                v = sbuf[b, r, pl.ds(col, _L)]
                f = fbuf[b, r, pl.ds(col, _L)]
                m = f > 0.1
                a_sum = a_sum + v
                a_sq = a_sq + v * v
                a_m = a_m + jnp.where(m, v, 0.0)
                a_c = a_c + jnp.where(m, 1.0, 0.0)
                out.append((a_sum, a_sq, a_m, a_c))
            return tuple(out)

        return plsc.parallel_loop(0, _VPC // _U, unroll=2, carry=accs)(step)

    accs = accs0
    for i in range(_CPW):
        b = i % 2
        for cp in copies(i, b):
            cp.wait()
        accs = chunk_compute(b, accs)
        if i + 2 < _CPW:
            for cp in copies(i + 2, b):
                cp.start()
    a_sum, a_sq, a_m, a_c = accs[0]
    for u in range(1, _U):
        a_sum = a_sum + accs[u][0]
        a_sq = a_sq + accs[u][1]
        a_m = a_m + accs[u][2]
        a_c = a_c + accs[u][3]

    part[0, :] = a_sum
    part[1, :] = a_sq
    part[2, :] = a_m
    part[3, :] = a_c
    pltpu.sync_copy(part, out_hbm.at[wid])


_sc_reduce = functools.partial(
    pl.kernel,
    mesh=plsc.VectorSubcoreMesh(core_axis_name="c", subcore_axis_name="s"),
    out_type=jax.ShapeDtypeStruct((_NW, 4, _L), jnp.float32),
    scratch_types=[
        pltpu.VMEM((2, _ROWS, _W), jnp.float32),
        pltpu.VMEM((2, _ROWS, _W), jnp.float32),
        pltpu.VMEM((4, _L), jnp.float32),
        pltpu.SemaphoreType.DMA,
        pltpu.SemaphoreType.DMA,
        pltpu.SemaphoreType.DMA,
        pltpu.SemaphoreType.DMA,
    ],
)(_sc_body)


def _tc_body(sal_ref, fix_ref, out_ref):
    i = pl.program_id(0) + pl.program_id(1)
    s = sal_ref[...]
    f = fix_ref[...]
    m = f > 0.1
    ssum = jnp.sum(s)
    ssq = jnp.sum(s * s)
    msum = jnp.sum(jnp.where(m, s, 0.0))
    cnt = jnp.sum(jnp.where(m, 1.0, 0.0))

    @pl.when(i == 0)
    def _init():
        out_ref[0] = 0.0
        out_ref[1] = 0.0
        out_ref[2] = 0.0
        out_ref[3] = 0.0

    out_ref[0] += ssum
    out_ref[1] += ssq
    out_ref[2] += msum
    out_ref[3] += cnt


def kernel(sal_map, fix):
    n = sal_map.size
    tc_partials = pl.pallas_call(
        _tc_body,
        grid=(_B // 4, 1),
        in_specs=[
            pl.BlockSpec((4, _H, _W), lambda i, j: (i, 0, 0)),
            pl.BlockSpec((4, _H, _W), lambda i, j: (i, 0, 0)),
        ],
        out_specs=pl.BlockSpec(memory_space=pltpu.SMEM),
        out_shape=jax.ShapeDtypeStruct((4,), jnp.float32),
    )(sal_map, fix)
    sums = tc_partials
    ssum, ssq, msum, cnt = sums[0], sums[1], sums[2], sums[3]
    nf = jnp.float32(n)
    mean = ssum / nf
    var = (ssq - nf * mean * mean) / (nf - 1.0)
    std = jnp.sqrt(var)
    return (msum - cnt * mean) / (std * cnt)
